# Initial kernel scaffold; baseline (speedup 1.0000x reference)
import jax
import jax.numpy as jnp
from jax.experimental import pallas as pl


def kernel(x, duration, max_len):
    raise NotImplementedError("write your pallas kernel here")


# speedup vs baseline: 36.9106x; 36.9106x over previous
import jax
import jax.numpy as jnp
from jax import lax
from jax.experimental import pallas as pl
from jax.experimental.pallas import tpu as pltpu
from jax.experimental.pallas import tpu_sc as plsc

NC, NS, L = 2, 16, 16
NW = NC * NS
B, T, D = 8, 512, 512
MAX_LEN = 4096
QW = 4
POS_W = MAX_LEN // QW
G = 64
NT = POS_W // G
ZROW = B * T


def _expand_body(xpad, dur, out, mel, dur_v, idxarr, idxg, gbuf, zbuf, melv, sem):
    wid = lax.axis_index("s") * NC + lax.axis_index("c")
    b = wid // QW
    q = wid % QW
    base = q * POS_W
    wbase = b * MAX_LEN + base

    pltpu.sync_copy(dur.at[b], dur_v)
    pltpu.sync_copy(xpad.at[pl.ds(ZROW, G)], zbuf)

    zv = jnp.zeros((L,), jnp.int32)

    def init_body(k, c):
        idxarr[pl.ds(k * L, L)] = zv
        return c
    lax.fori_loop(0, MAX_LEN // L, init_body, 0)

    def p1(k, csum_base):
        v = dur_v[pl.ds(k * L, L)]
        c = plsc.cumsum(v) + csum_base
        start = c - v
        t = lax.iota(jnp.int32, L) + k * L
        m = (v > 0) & (start < MAX_LEN)
        plsc.store_scatter(idxarr, [start], t, mask=m)
        return jnp.max(c)
    total = lax.fori_loop(0, T // L, p1, jnp.int32(0))

    def p2a(k, mv):
        return jnp.maximum(mv, idxarr[pl.ds(k * L, L)])
    m0 = jnp.max(lax.fori_loop(0, q * (POS_W // L), p2a, zv))

    def p2b(k, m):
        kk = q * (POS_W // L) + k
        v = idxarr[pl.ds(kk * L, L)]
        c = jnp.maximum(plsc.cummax(v), m)
        p = kk * L + lax.iota(jnp.int32, L)
        idxg[pl.ds(k * L, L)] = jnp.where(p < total, b * T + c, ZROW)
        return jnp.max(c)
    lax.fori_loop(0, POS_W // L, p2b, m0)

    nv = jnp.clip(total - base, 0, POS_W)
    ntiles = (nv + (G - 1)) // G

    def gather_tile(j, c):
        pltpu.async_copy(xpad.at[idxg.at[pl.ds(j * G, G)]], gbuf, sem).wait()
        pltpu.sync_copy(gbuf, out.at[pl.ds(wbase + j * G, G)])
        return c
    lax.fori_loop(0, ntiles, gather_tile, 0)

    def zero_tile(j, c):
        pltpu.sync_copy(zbuf, out.at[pl.ds(wbase + j * G, G)])
        return c
    lax.fori_loop(ntiles, NT, zero_tile, 0)

    @pl.when(q == 0)
    def _():
        melv[...] = jnp.broadcast_to(total, (L,))
        pltpu.sync_copy(melv, mel.at[b])


_sc_expand = pl.kernel(
    _expand_body,
    out_type=(jax.ShapeDtypeStruct((B * MAX_LEN, D), jnp.float32),
              jax.ShapeDtypeStruct((B, L), jnp.int32)),
    mesh=plsc.VectorSubcoreMesh(core_axis_name="c", subcore_axis_name="s",
                                num_cores=NC, num_subcores=NS),
    compiler_params=pltpu.CompilerParams(needs_layout_passes=False),
    scratch_types=[
        pltpu.VMEM((T,), jnp.int32),
        pltpu.VMEM((MAX_LEN,), jnp.int32),
        pltpu.VMEM((POS_W,), jnp.int32),
        pltpu.VMEM((G, D), jnp.float32),
        pltpu.VMEM((G, D), jnp.float32),
        pltpu.VMEM((L,), jnp.int32),
        pltpu.SemaphoreType.DMA,
    ],
)


def kernel(x, duration, max_len):
    xflat = x.reshape(B * T, D)
    xpad = jnp.concatenate([xflat, jnp.zeros((G, D), x.dtype)], axis=0)
    outflat, mel16 = _sc_expand(xpad, duration)
    return outflat.reshape(B, MAX_LEN, D), mel16[:, 0]


# speedup vs baseline: 45.9988x; 1.2462x over previous
import jax
import jax.numpy as jnp
from jax import lax
from jax.experimental import pallas as pl
from jax.experimental.pallas import tpu as pltpu
from jax.experimental.pallas import tpu_sc as plsc

NC, NS, L = 2, 16, 16
NW = NC * NS
B, T, D = 8, 512, 512
MAX_LEN = 4096
QW = 4
POS_W = MAX_LEN // QW
G = 64
NT = POS_W // G
ZROW = B * T


def _expand_body(xpad, dur, out, mel,
                 dur_v, idxarr, idxg, gbuf, zbuf, melv, gsem, wsem, zsem):
    wid = lax.axis_index("s") * NC + lax.axis_index("c")
    b = wid // QW
    q = wid % QW
    base = q * POS_W
    wbase = b * MAX_LEN + base

    pltpu.sync_copy(dur.at[b], dur_v)
    zcp = pltpu.make_async_copy(xpad.at[pl.ds(ZROW, G)], zbuf, zsem)
    zcp.start()

    zv = jnp.zeros((L,), jnp.int32)

    def init_body(k, c):
        idxarr[pl.ds(k * L, L)] = zv
        return c
    lax.fori_loop(0, MAX_LEN // L, init_body, 0)

    def p1(k, csum_base):
        v = dur_v[pl.ds(k * L, L)]
        c = plsc.cumsum(v) + csum_base
        start = c - v
        t = lax.iota(jnp.int32, L) + k * L
        m = (v > 0) & (start < MAX_LEN)
        plsc.store_scatter(idxarr, [start], t, mask=m)
        return jnp.max(c)
    total = lax.fori_loop(0, T // L, p1, jnp.int32(0))

    nv = jnp.clip(total - base, 0, POS_W)
    ng = (nv + (G - 1)) // G

    zcp.wait()
    def zfire(j, c):
        pltpu.async_copy(zbuf, out.at[pl.ds(wbase + j * G, G)], zsem)
        return c
    lax.fori_loop(ng, NT, zfire, 0)

    def p2a(k, mv):
        return jnp.maximum(mv, idxarr[pl.ds(k * L, L)])
    m0 = jnp.max(lax.fori_loop(0, q * (POS_W // L), p2a, zv))

    def p2b(k, m):
        kk = q * (POS_W // L) + k
        v = idxarr[pl.ds(kk * L, L)]
        c = jnp.maximum(plsc.cummax(v), m)
        p = kk * L + lax.iota(jnp.int32, L)
        idxg[pl.ds(k * L, L)] = jnp.where(p < total, b * T + c, ZROW)
        return jnp.max(c)
    lax.fori_loop(0, POS_W // L, p2b, m0)

    def gstart(j, s):
        pltpu.async_copy(xpad.at[idxg.at[pl.ds(j * G, G)]],
                         gbuf.at[s], gsem.at[s])

    @pl.when(ng > 0)
    def _():
        gstart(0, 0)

    def body(j, c):
        cur = j % 2
        nxt = (j + 1) % 2

        @pl.when(j + 1 < ng)
        def _():
            @pl.when(j >= 1)
            def _():
                pltpu.make_async_copy(
                    gbuf.at[nxt], out.at[pl.ds(wbase, G)], wsem.at[nxt]).wait()
            gstart(j + 1, nxt)

        pltpu.make_async_copy(
            xpad.at[pl.ds(0, G)], gbuf.at[cur], gsem.at[cur]).wait()
        pltpu.async_copy(gbuf.at[cur], out.at[pl.ds(wbase + j * G, G)],
                         wsem.at[cur])
        return c
    lax.fori_loop(0, ng, body, 0)

    @pl.when(ng >= 2)
    def _():
        pltpu.make_async_copy(
            gbuf.at[ng % 2], out.at[pl.ds(wbase, G)], wsem.at[ng % 2]).wait()

    @pl.when(ng >= 1)
    def _():
        s = (ng + 1) % 2
        pltpu.make_async_copy(
            gbuf.at[s], out.at[pl.ds(wbase, G)], wsem.at[s]).wait()

    def zdrain(j, c):
        pltpu.make_async_copy(zbuf, out.at[pl.ds(wbase, G)], zsem).wait()
        return c
    lax.fori_loop(ng, NT, zdrain, 0)

    @pl.when(q == 0)
    def _():
        melv[...] = jnp.broadcast_to(total, (L,))
        pltpu.sync_copy(melv, mel.at[b])


_sc_expand = pl.kernel(
    _expand_body,
    out_type=(jax.ShapeDtypeStruct((B * MAX_LEN, D), jnp.float32),
              jax.ShapeDtypeStruct((B, L), jnp.int32)),
    mesh=plsc.VectorSubcoreMesh(core_axis_name="c", subcore_axis_name="s",
                                num_cores=NC, num_subcores=NS),
    compiler_params=pltpu.CompilerParams(needs_layout_passes=False),
    scratch_types=[
        pltpu.VMEM((T,), jnp.int32),
        pltpu.VMEM((MAX_LEN,), jnp.int32),
        pltpu.VMEM((POS_W,), jnp.int32),
        pltpu.VMEM((2, G, D), jnp.float32),
        pltpu.VMEM((G, D), jnp.float32),
        pltpu.VMEM((L,), jnp.int32),
        pltpu.SemaphoreType.DMA((2,)),
        pltpu.SemaphoreType.DMA((2,)),
        pltpu.SemaphoreType.DMA,
    ],
)


def kernel(x, duration, max_len):
    xflat = x.reshape(B * T, D)
    xpad = jnp.concatenate([xflat, jnp.zeros((G, D), x.dtype)], axis=0)
    outflat, mel16 = _sc_expand(xpad, duration)
    return outflat.reshape(B, MAX_LEN, D), mel16[:, 0]


# speedup vs baseline: 48.0039x; 1.0436x over previous
import jax
import jax.numpy as jnp
from jax import lax
from jax.experimental import pallas as pl
from jax.experimental.pallas import tpu as pltpu
from jax.experimental.pallas import tpu_sc as plsc

NC, NS, L = 2, 16, 16
NW = NC * NS
B, T, D = 8, 512, 512
MAX_LEN = 4096
QW = 4
POS_W = MAX_LEN // QW
G = 64
NT = POS_W // G


def _expand_body(xflat, dur, out, mel,
                 dur_v, idxarr, idxg, gbuf, zbuf, melv, gsem, wsem, zsem):
    wid = lax.axis_index("s") * NC + lax.axis_index("c")
    b = wid // QW
    q = wid % QW
    base = q * POS_W
    wbase = b * MAX_LEN + base

    pltpu.sync_copy(dur.at[b], dur_v)

    zv = jnp.zeros((L,), jnp.int32)
    zvf = jnp.zeros((L,), jnp.float32)

    def init_body(k, c):
        idxarr[pl.ds(k * L, L)] = zv
        return c
    lax.fori_loop(0, MAX_LEN // L, init_body, 0)

    def zb_body(r, c):
        for kk in range(D // L):
            zbuf[r, pl.ds(kk * L, L)] = zvf
        return c
    lax.fori_loop(0, G, zb_body, 0)

    def p1(k, csum_base):
        v = dur_v[pl.ds(k * L, L)]
        c = plsc.cumsum(v) + csum_base
        start = c - v
        t = lax.iota(jnp.int32, L) + k * L
        m = (v > 0) & (start < MAX_LEN)
        plsc.store_scatter(idxarr, [start], t, mask=m)
        return jnp.max(c)
    total = lax.fori_loop(0, T // L, p1, jnp.int32(0))

    nv = jnp.clip(total - base, 0, POS_W)
    ng = (nv + (G - 1)) // G
    pt = nv - (ng - 1) * G

    def zfire(j, c):
        pltpu.async_copy(zbuf, out.at[pl.ds(wbase + j * G, G)], zsem)
        return c
    lax.fori_loop(ng, NT, zfire, 0)

    def p2a(k, mv):
        return jnp.maximum(mv, idxarr[pl.ds(k * L, L)])
    m0 = jnp.max(lax.fori_loop(0, q * (POS_W // L), p2a, zv))

    def p2b(k, m):
        kk = q * (POS_W // L) + k
        v = idxarr[pl.ds(kk * L, L)]
        c = jnp.maximum(plsc.cummax(v), m)
        p = kk * L + lax.iota(jnp.int32, L)
        idxg[pl.ds(k * L, L)] = jnp.where(p < total, b * T + c, 0)
        return jnp.max(c)
    lax.fori_loop(0, POS_W // L, p2b, m0)

    def gstart(j, s):
        pltpu.async_copy(xflat.at[idxg.at[pl.ds(j * G, G)]],
                         gbuf.at[s], gsem.at[s])

    @pl.when(ng > 0)
    def _():
        gstart(0, 0)

    def body(j, c):
        cur = j % 2
        nxt = (j + 1) % 2

        @pl.when(j + 1 < ng)
        def _():
            @pl.when(j >= 1)
            def _():
                pltpu.make_async_copy(
                    gbuf.at[nxt], out.at[pl.ds(wbase, G)], wsem.at[nxt]).wait()
            gstart(j + 1, nxt)

        pltpu.make_async_copy(
            xflat.at[pl.ds(0, G)], gbuf.at[cur], gsem.at[cur]).wait()

        @pl.when((j == ng - 1) & (pt < G))
        def _():
            def zr(r, c2):
                for kk in range(D // L):
                    gbuf[cur, r, pl.ds(kk * L, L)] = zvf
                return c2
            lax.fori_loop(pt, G, zr, 0)

        pltpu.async_copy(gbuf.at[cur], out.at[pl.ds(wbase + j * G, G)],
                         wsem.at[cur])
        return c
    lax.fori_loop(0, ng, body, 0)

    @pl.when(ng >= 2)
    def _():
        pltpu.make_async_copy(
            gbuf.at[ng % 2], out.at[pl.ds(wbase, G)], wsem.at[ng % 2]).wait()

    @pl.when(ng >= 1)
    def _():
        s = (ng + 1) % 2
        pltpu.make_async_copy(
            gbuf.at[s], out.at[pl.ds(wbase, G)], wsem.at[s]).wait()

    def zdrain(j, c):
        pltpu.make_async_copy(zbuf, out.at[pl.ds(wbase, G)], zsem).wait()
        return c
    lax.fori_loop(ng, NT, zdrain, 0)

    @pl.when(q == 0)
    def _():
        melv[...] = jnp.broadcast_to(total, (L,))
        pltpu.sync_copy(melv, mel.at[b])


_sc_expand = pl.kernel(
    _expand_body,
    out_type=(jax.ShapeDtypeStruct((B * MAX_LEN, D), jnp.float32),
              jax.ShapeDtypeStruct((B, L), jnp.int32)),
    mesh=plsc.VectorSubcoreMesh(core_axis_name="c", subcore_axis_name="s",
                                num_cores=NC, num_subcores=NS),
    compiler_params=pltpu.CompilerParams(needs_layout_passes=False),
    scratch_types=[
        pltpu.VMEM((T,), jnp.int32),
        pltpu.VMEM((MAX_LEN,), jnp.int32),
        pltpu.VMEM((POS_W,), jnp.int32),
        pltpu.VMEM((2, G, D), jnp.float32),
        pltpu.VMEM((G, D), jnp.float32),
        pltpu.VMEM((L,), jnp.int32),
        pltpu.SemaphoreType.DMA((2,)),
        pltpu.SemaphoreType.DMA((2,)),
        pltpu.SemaphoreType.DMA,
    ],
)


def kernel(x, duration, max_len):
    outflat, mel16 = _sc_expand(x.reshape(B * T, D), duration)
    return outflat.reshape(B, MAX_LEN, D), mel16[:, 0]


# speedup vs baseline: 50.8641x; 1.0596x over previous
import jax
import jax.numpy as jnp
from jax import lax
from jax.experimental import pallas as pl
from jax.experimental.pallas import tpu as pltpu
from jax.experimental.pallas import tpu_sc as plsc

NC, NS, L = 2, 16, 16
NW = NC * NS
B, T, D = 8, 512, 512
MAX_LEN = 4096
QW = 4
G = 64
NTB = MAX_LEN // G
GPL = G // L
MAXG = (NTB + QW - 1) // QW


def _expand_body(xflat, dur, out, mel,
                 dur_v, idxarr, idxg, gbuf, zbuf, melv, gsem, wsem, zsem):
    wid = lax.axis_index("s") * NC + lax.axis_index("c")
    b = wid // QW
    q = wid % QW
    obase = b * MAX_LEN

    pltpu.sync_copy(dur.at[b], dur_v)

    zv = jnp.zeros((L,), jnp.int32)
    zvf = jnp.zeros((L,), jnp.float32)

    def init_body(k, c):
        idxarr[pl.ds(k * L, L)] = zv
        return c
    lax.fori_loop(0, MAX_LEN // L, init_body, 0)

    def zb_body(r, c):
        for kk in range(D // L):
            zbuf[r, pl.ds(kk * L, L)] = zvf
        return c
    lax.fori_loop(0, G, zb_body, 0)

    def p1(k, csum_base):
        v = dur_v[pl.ds(k * L, L)]
        c = plsc.cumsum(v) + csum_base
        start = c - v
        t = lax.iota(jnp.int32, L) + k * L
        m = (v > 0) & (start < MAX_LEN)
        plsc.store_scatter(idxarr, [start], t, mask=m)
        return jnp.max(c)
    total = lax.fori_loop(0, T // L, p1, jnp.int32(0))

    nvt = (total + (G - 1)) // G
    nzt = NTB - nvt
    gs = q * nvt // QW
    ge = (q + 1) * nvt // QW
    ng = ge - gs
    zs = nvt + q * nzt // QW
    ze = nvt + (q + 1) * nzt // QW

    def zfire(j, c):
        pltpu.async_copy(zbuf, out.at[pl.ds(obase + j * G, G)], zsem)
        return c
    lax.fori_loop(zs, ze, zfire, 0)

    def p2a(k, mv):
        return jnp.maximum(mv, idxarr[pl.ds(k * L, L)])
    m0 = jnp.max(lax.fori_loop(0, gs * GPL, p2a, zv))

    def p2b(k, m):
        kk = gs * GPL + k
        v = idxarr[pl.ds(kk * L, L)]
        c = jnp.maximum(plsc.cummax(v), m)
        p = kk * L + lax.iota(jnp.int32, L)
        idxg[pl.ds(k * L, L)] = jnp.where(p < total, b * T + c, 0)
        return jnp.max(c)
    lax.fori_loop(0, ng * GPL, p2b, m0)

    def gstart(jl, s):
        pltpu.async_copy(xflat.at[idxg.at[pl.ds(jl * G, G)]],
                         gbuf.at[s], gsem.at[s])

    @pl.when(ng > 0)
    def _():
        gstart(0, 0)

    def body(jl, c):
        cur = jl % 2
        nxt = (jl + 1) % 2

        @pl.when(jl + 1 < ng)
        def _():
            @pl.when(jl >= 1)
            def _():
                pltpu.make_async_copy(
                    gbuf.at[nxt], out.at[pl.ds(obase, G)], wsem.at[nxt]).wait()
            gstart(jl + 1, nxt)

        pltpu.make_async_copy(
            xflat.at[pl.ds(0, G)], gbuf.at[cur], gsem.at[cur]).wait()

        vt = jnp.clip(total - (gs + jl) * G, 0, G)

        @pl.when(vt < G)
        def _():
            def zr(r, c2):
                for kk in range(D // L):
                    gbuf[cur, r, pl.ds(kk * L, L)] = zvf
                return c2
            lax.fori_loop(vt, G, zr, 0)

        pltpu.async_copy(gbuf.at[cur], out.at[pl.ds(obase + (gs + jl) * G, G)],
                         wsem.at[cur])
        return c
    lax.fori_loop(0, ng, body, 0)

    @pl.when(ng >= 2)
    def _():
        pltpu.make_async_copy(
            gbuf.at[ng % 2], out.at[pl.ds(obase, G)], wsem.at[ng % 2]).wait()

    @pl.when(ng >= 1)
    def _():
        s = (ng + 1) % 2
        pltpu.make_async_copy(
            gbuf.at[s], out.at[pl.ds(obase, G)], wsem.at[s]).wait()

    def zdrain(j, c):
        pltpu.make_async_copy(zbuf, out.at[pl.ds(obase, G)], zsem).wait()
        return c
    lax.fori_loop(zs, ze, zdrain, 0)

    @pl.when(q == 0)
    def _():
        melv[...] = jnp.broadcast_to(total, (L,))
        pltpu.sync_copy(melv, mel.at[b])


_sc_expand = pl.kernel(
    _expand_body,
    out_type=(jax.ShapeDtypeStruct((B * MAX_LEN, D), jnp.float32),
              jax.ShapeDtypeStruct((B, L), jnp.int32)),
    mesh=plsc.VectorSubcoreMesh(core_axis_name="c", subcore_axis_name="s",
                                num_cores=NC, num_subcores=NS),
    compiler_params=pltpu.CompilerParams(needs_layout_passes=False),
    scratch_types=[
        pltpu.VMEM((T,), jnp.int32),
        pltpu.VMEM((MAX_LEN,), jnp.int32),
        pltpu.VMEM((MAXG * G,), jnp.int32),
        pltpu.VMEM((2, G, D), jnp.float32),
        pltpu.VMEM((G, D), jnp.float32),
        pltpu.VMEM((L,), jnp.int32),
        pltpu.SemaphoreType.DMA((2,)),
        pltpu.SemaphoreType.DMA((2,)),
        pltpu.SemaphoreType.DMA,
    ],
)


def kernel(x, duration, max_len):
    outflat, mel16 = _sc_expand(x.reshape(B * T, D), duration)
    return outflat.reshape(B, MAX_LEN, D), mel16[:, 0]


# speedup vs baseline: 51.1288x; 1.0052x over previous
import jax
import jax.numpy as jnp
from jax import lax
from jax.experimental import pallas as pl
from jax.experimental.pallas import tpu as pltpu
from jax.experimental.pallas import tpu_sc as plsc

NC, NS, L = 2, 16, 16
NW = NC * NS
B, T, D = 8, 512, 512
MAX_LEN = 4096
QW = 4
G = 64
NTB = MAX_LEN // G
GPL = G // L
MAXG = (NTB + QW - 1) // QW
ZR = 16


def _expand_body(xflat, dur, out, mel,
                 dur_v, idxarr, idxg, gbuf, zbuf, melv, gsem, wsem, zsem):
    wid = lax.axis_index("s") * NC + lax.axis_index("c")
    b = wid // QW
    q = (wid + b) % QW
    obase = b * MAX_LEN

    pltpu.sync_copy(dur.at[b], dur_v)

    zv = jnp.zeros((L,), jnp.int32)
    zvf = jnp.zeros((L,), jnp.float32)

    def init_body(k, c):
        idxarr[pl.ds(k * L, L)] = zv
        return c
    lax.fori_loop(0, MAX_LEN // L, init_body, 0)

    def zb_body(r, c):
        for kk in range(D // L):
            zbuf[r, pl.ds(kk * L, L)] = zvf
        return c
    lax.fori_loop(0, ZR, zb_body, 0)

    def p1(k, csum_base):
        v = dur_v[pl.ds(k * L, L)]
        c = plsc.cumsum(v) + csum_base
        start = c - v
        t = lax.iota(jnp.int32, L) + k * L
        m = (v > 0) & (start < MAX_LEN)
        plsc.store_scatter(idxarr, [start], t, mask=m)
        return jnp.max(c)
    total = lax.fori_loop(0, T // L, p1, jnp.int32(0))

    nvt = (total + (G - 1)) // G
    nzt = NTB - nvt
    gs = q * nvt // QW
    ge = (q + 1) * nvt // QW
    ng = ge - gs
    zs = nvt + q * nzt // QW
    ze = nvt + (q + 1) * nzt // QW

    def zfire(j, c):
        for h in range(G // ZR):
            pltpu.async_copy(zbuf,
                             out.at[pl.ds(obase + j * G + h * ZR, ZR)], zsem)
        return c
    lax.fori_loop(zs, ze, zfire, 0)

    def p2a(k, mv):
        return jnp.maximum(mv, idxarr[pl.ds(k * L, L)])
    m0 = jnp.max(lax.fori_loop(0, gs * GPL, p2a, zv))

    def p2b(k, m):
        kk = gs * GPL + k
        v = idxarr[pl.ds(kk * L, L)]
        c = jnp.maximum(plsc.cummax(v), m)
        p = kk * L + lax.iota(jnp.int32, L)
        idxg[pl.ds(k * L, L)] = jnp.where(p < total, b * T + c, 0)
        return jnp.max(c)
    lax.fori_loop(0, ng * GPL, p2b, m0)

    def gstart(jl, s):
        pltpu.async_copy(xflat.at[idxg.at[pl.ds(jl * G, G)]],
                         gbuf.at[s], gsem.at[s])

    @pl.when(ng > 0)
    def _():
        gstart(0, 0)

    def body(jl, c):
        cur = jl % 2
        nxt = (jl + 1) % 2

        @pl.when(jl + 1 < ng)
        def _():
            @pl.when(jl >= 1)
            def _():
                pltpu.make_async_copy(
                    gbuf.at[nxt], out.at[pl.ds(obase, G)], wsem.at[nxt]).wait()
            gstart(jl + 1, nxt)

        pltpu.make_async_copy(
            xflat.at[pl.ds(0, G)], gbuf.at[cur], gsem.at[cur]).wait()

        vt = jnp.clip(total - (gs + jl) * G, 0, G)

        @pl.when(vt < G)
        def _():
            def zr(r, c2):
                for kk in range(D // L):
                    gbuf[cur, r, pl.ds(kk * L, L)] = zvf
                return c2
            lax.fori_loop(vt, G, zr, 0)

        pltpu.async_copy(gbuf.at[cur], out.at[pl.ds(obase + (gs + jl) * G, G)],
                         wsem.at[cur])
        return c
    lax.fori_loop(0, ng, body, 0)

    @pl.when(ng >= 2)
    def _():
        pltpu.make_async_copy(
            gbuf.at[ng % 2], out.at[pl.ds(obase, G)], wsem.at[ng % 2]).wait()

    @pl.when(ng >= 1)
    def _():
        s = (ng + 1) % 2
        pltpu.make_async_copy(
            gbuf.at[s], out.at[pl.ds(obase, G)], wsem.at[s]).wait()

    def zdrain(j, c):
        pltpu.make_async_copy(zbuf, out.at[pl.ds(obase, ZR)], zsem).wait()
        return c
    lax.fori_loop(0, (ze - zs) * (G // ZR), zdrain, 0)

    @pl.when(q == 0)
    def _():
        melv[...] = jnp.broadcast_to(total, (L,))
        pltpu.sync_copy(melv, mel.at[b])


_sc_expand = pl.kernel(
    _expand_body,
    out_type=(jax.ShapeDtypeStruct((B * MAX_LEN, D), jnp.float32),
              jax.ShapeDtypeStruct((B, L), jnp.int32)),
    mesh=plsc.VectorSubcoreMesh(core_axis_name="c", subcore_axis_name="s",
                                num_cores=NC, num_subcores=NS),
    compiler_params=pltpu.CompilerParams(needs_layout_passes=False),
    scratch_types=[
        pltpu.VMEM((T,), jnp.int32),
        pltpu.VMEM((MAX_LEN,), jnp.int32),
        pltpu.VMEM((MAXG * G,), jnp.int32),
        pltpu.VMEM((2, G, D), jnp.float32),
        pltpu.VMEM((ZR, D), jnp.float32),
        pltpu.VMEM((L,), jnp.int32),
        pltpu.SemaphoreType.DMA((2,)),
        pltpu.SemaphoreType.DMA((2,)),
        pltpu.SemaphoreType.DMA,
    ],
)


def kernel(x, duration, max_len):
    outflat, mel16 = _sc_expand(x.reshape(B * T, D), duration)
    return outflat.reshape(B, MAX_LEN, D), mel16[:, 0]


# speedup vs baseline: 51.3313x; 1.0040x over previous
import jax
import jax.numpy as jnp
from jax import lax
from jax.experimental import pallas as pl
from jax.experimental.pallas import tpu as pltpu
from jax.experimental.pallas import tpu_sc as plsc

NC, NS, L = 2, 16, 16
NW = NC * NS
B, T, D = 8, 512, 512
MAX_LEN = 4096
QW = 4
G = 64
NTB = MAX_LEN // G
GPL = G // L
MAXG = (NTB + QW - 1) // QW
ZR = 16
NB = 3


def _expand_body(xflat, dur, out, mel,
                 dur_v, idxarr, idxg, gbuf, zbuf, melv, gsem, wsem, zsem):
    wid = lax.axis_index("s") * NC + lax.axis_index("c")
    b = wid // QW
    q = (wid + b) % QW
    obase = b * MAX_LEN

    pltpu.sync_copy(dur.at[b], dur_v)

    zv = jnp.zeros((L,), jnp.int32)
    zvf = jnp.zeros((L,), jnp.float32)

    def init_body(k, c):
        idxarr[pl.ds(k * L, L)] = zv
        return c
    lax.fori_loop(0, MAX_LEN // L, init_body, 0)

    def zb_body(r, c):
        for kk in range(D // L):
            zbuf[r, pl.ds(kk * L, L)] = zvf
        return c
    lax.fori_loop(0, ZR, zb_body, 0)

    def p1(k, csum_base):
        v = dur_v[pl.ds(k * L, L)]
        c = plsc.cumsum(v) + csum_base
        start = c - v
        t = lax.iota(jnp.int32, L) + k * L
        m = (v > 0) & (start < MAX_LEN)
        plsc.store_scatter(idxarr, [start], t, mask=m)
        return jnp.max(c)
    total = lax.fori_loop(0, T // L, p1, jnp.int32(0))

    nvt = (total + (G - 1)) // G
    nzt = NTB - nvt
    gs = q * nvt // QW
    ge = (q + 1) * nvt // QW
    ng = ge - gs
    zs = nvt + q * nzt // QW
    ze = nvt + (q + 1) * nzt // QW

    def zfire(j, c):
        for h in range(G // ZR):
            pltpu.async_copy(zbuf,
                             out.at[pl.ds(obase + j * G + h * ZR, ZR)], zsem)
        return c
    lax.fori_loop(zs, ze, zfire, 0)

    def p2a(k, mv):
        return jnp.maximum(mv, idxarr[pl.ds(k * L, L)])
    m0 = jnp.max(lax.fori_loop(0, gs * GPL, p2a, zv))

    def p2b(k, m):
        kk = gs * GPL + k
        v = idxarr[pl.ds(kk * L, L)]
        c = jnp.maximum(plsc.cummax(v), m)
        p = kk * L + lax.iota(jnp.int32, L)
        idxg[pl.ds(k * L, L)] = jnp.where(p < total, b * T + c, 0)
        return jnp.max(c)
    lax.fori_loop(0, ng * GPL, p2b, m0)

    def gstart(jl):
        s = jl % NB
        pltpu.async_copy(xflat.at[idxg.at[pl.ds(jl * G, G)]],
                         gbuf.at[s], gsem.at[s])

    def prime(i, c):
        gstart(i)
        return c
    lax.fori_loop(0, jnp.minimum(NB, ng), prime, 0)

    def body(jl, c):
        cur = jl % NB

        @pl.when((jl >= 1) & (jl + NB - 1 < ng))
        def _():
            pltpu.make_async_copy(
                gbuf.at[(jl - 1) % NB], out.at[pl.ds(obase, G)],
                wsem.at[(jl - 1) % NB]).wait()
            gstart(jl + NB - 1)

        pltpu.make_async_copy(
            xflat.at[pl.ds(0, G)], gbuf.at[cur], gsem.at[cur]).wait()

        vt = jnp.clip(total - (gs + jl) * G, 0, G)

        @pl.when(vt < G)
        def _():
            def zr(r, c2):
                for kk in range(D // L):
                    gbuf[cur, r, pl.ds(kk * L, L)] = zvf
                return c2
            lax.fori_loop(vt, G, zr, 0)

        pltpu.async_copy(gbuf.at[cur], out.at[pl.ds(obase + (gs + jl) * G, G)],
                         wsem.at[cur])
        return c
    lax.fori_loop(0, ng, body, 0)

    def wdrain(i, c):
        pltpu.make_async_copy(
            gbuf.at[i % NB], out.at[pl.ds(obase, G)], wsem.at[i % NB]).wait()
        return c
    lax.fori_loop(jnp.maximum(ng - NB, 0), ng, wdrain, 0)

    def zdrain(j, c):
        pltpu.make_async_copy(zbuf, out.at[pl.ds(obase, ZR)], zsem).wait()
        return c
    lax.fori_loop(0, (ze - zs) * (G // ZR), zdrain, 0)

    @pl.when(q == 0)
    def _():
        melv[...] = jnp.broadcast_to(total, (L,))
        pltpu.sync_copy(melv, mel.at[b])


_sc_expand = pl.kernel(
    _expand_body,
    out_type=(jax.ShapeDtypeStruct((B * MAX_LEN, D), jnp.float32),
              jax.ShapeDtypeStruct((B, L), jnp.int32)),
    mesh=plsc.VectorSubcoreMesh(core_axis_name="c", subcore_axis_name="s",
                                num_cores=NC, num_subcores=NS),
    compiler_params=pltpu.CompilerParams(needs_layout_passes=False),
    scratch_types=[
        pltpu.VMEM((T,), jnp.int32),
        pltpu.VMEM((MAX_LEN,), jnp.int32),
        pltpu.VMEM((MAXG * G,), jnp.int32),
        pltpu.VMEM((NB, G, D), jnp.float32),
        pltpu.VMEM((ZR, D), jnp.float32),
        pltpu.VMEM((L,), jnp.int32),
        pltpu.SemaphoreType.DMA((NB,)),
        pltpu.SemaphoreType.DMA((NB,)),
        pltpu.SemaphoreType.DMA,
    ],
)


def kernel(x, duration, max_len):
    outflat, mel16 = _sc_expand(x.reshape(B * T, D), duration)
    return outflat.reshape(B, MAX_LEN, D), mel16[:, 0]
